# Initial kernel scaffold; baseline (speedup 1.0000x reference)
import jax
import jax.numpy as jnp
from jax.experimental import pallas as pl


def kernel(features, edge_index, W1a, b1a, W2a, b2a, eps1, W1b, b1b, W2b, b2b, eps2):
    raise NotImplementedError("write your pallas kernel here")


# speedup vs baseline: 5.4144x; 5.4144x over previous
import functools

import jax
import jax.numpy as jnp
from jax import lax
from jax.experimental import pallas as pl
from jax.experimental.pallas import tpu as pltpu
from jax.experimental.pallas import tpu_sc as plsc

N = 10000
D = 128
E = 320000

NC = 2
NS = 16
NW = NC * NS

CHUNK = 128
NCHUNKS = E // CHUNK
ROWS_A = 624
ROWS_REM = N - NS * ROWS_A
ZROWS = 156


def _sc_partial_segment_sum(x, edge_index):
    mesh = plsc.VectorSubcoreMesh(core_axis_name="c", subcore_axis_name="s")

    @functools.partial(
        pl.kernel,
        out_type=jax.ShapeDtypeStruct((NC, N, D), jnp.float32),
        mesh=mesh,
        scratch_types=[
            pltpu.VMEM_SHARED((N, D), jnp.float32),
            pltpu.VMEM((ZROWS, D), jnp.float32),
            pltpu.VMEM((CHUNK,), jnp.int32),
            pltpu.VMEM((CHUNK,), jnp.int32),
            pltpu.VMEM((CHUNK, D), jnp.float32),
            pltpu.SemaphoreType.DMA,
        ],
    )
    def k(x_hbm, ei_hbm, out_hbm, acc, zbuf, srcbuf, dstbuf, rows, sem):
        c = lax.axis_index("c")
        s = lax.axis_index("s")
        w = c * NS + s

        zeros16 = jnp.zeros((16,), jnp.float32)

        @pl.loop(0, ZROWS)
        def _(r):
            @pl.loop(0, D, step=16)
            def _(cc):
                zbuf[r, pl.ds(cc, 16)] = zeros16

        @pl.loop(0, ROWS_A, step=ZROWS)
        def _(r0):
            pltpu.sync_copy(zbuf, acc.at[pl.ds(s * ROWS_A + r0, ZROWS)])

        @pl.when(s == 0)
        def _():
            pltpu.sync_copy(zbuf.at[pl.ds(0, ROWS_REM)],
                            acc.at[pl.ds(NS * ROWS_A, ROWS_REM)])

        plsc.subcore_barrier()

        lo = w * NCHUNKS // NW
        hi = (w + 1) * NCHUNKS // NW

        @pl.loop(lo, hi)
        def _(j):
            base = j * CHUNK
            pltpu.sync_copy(ei_hbm.at[0, pl.ds(base, CHUNK)], srcbuf)
            pltpu.sync_copy(ei_hbm.at[1, pl.ds(base, CHUNK)], dstbuf)
            pltpu.async_copy(x_hbm.at[srcbuf], rows, sem).wait()
            pltpu.sync_copy(rows, acc.at[dstbuf], add=True)

        plsc.subcore_barrier()

        pltpu.sync_copy(
            acc.at[pl.ds(s * ROWS_A, ROWS_A)],
            out_hbm.at[c, pl.ds(s * ROWS_A, ROWS_A)],
        )

        @pl.when(s == 0)
        def _():
            pltpu.sync_copy(
                acc.at[pl.ds(NS * ROWS_A, ROWS_REM)],
                out_hbm.at[c, pl.ds(NS * ROWS_A, ROWS_REM)],
            )

    return k(x, edge_index)


def _tc_mlp_body(eps_ref, x_ref, a0_ref, a1_ref, w1_ref, b1_ref, w2_ref,
                 b2_ref, o_ref, *, relu_out):
    h = (1.0 + eps_ref[0, 0]) * x_ref[...] + a0_ref[...] + a1_ref[...]
    h = jnp.dot(h, w1_ref[...], preferred_element_type=jnp.float32,
                precision=lax.Precision.HIGHEST)
    h = jnp.maximum(h + b1_ref[...], 0.0)
    o = jnp.dot(h, w2_ref[...], preferred_element_type=jnp.float32,
                precision=lax.Precision.HIGHEST)
    o = o + b2_ref[...]
    if relu_out:
        o = jnp.maximum(o, 0.0)
    o_ref[...] = o


def _tc_gin_mlp(x, agg_partials, w1, b1, w2, b2, eps, relu_out):
    blk = 1000
    body = functools.partial(_tc_mlp_body, relu_out=relu_out)
    return pl.pallas_call(
        body,
        grid=(N // blk,),
        in_specs=[
            pl.BlockSpec((1, 1), lambda i: (0, 0), memory_space=pltpu.SMEM),
            pl.BlockSpec((blk, D), lambda i: (i, 0)),
            pl.BlockSpec((blk, D), lambda i: (i, 0)),
            pl.BlockSpec((blk, D), lambda i: (i, 0)),
            pl.BlockSpec((D, D), lambda i: (0, 0)),
            pl.BlockSpec((1, D), lambda i: (0, 0)),
            pl.BlockSpec((D, D), lambda i: (0, 0)),
            pl.BlockSpec((1, D), lambda i: (0, 0)),
        ],
        out_specs=pl.BlockSpec((blk, D), lambda i: (i, 0)),
        out_shape=jax.ShapeDtypeStruct((N, D), jnp.float32),
    )(
        eps.reshape(1, 1), x, agg_partials[0], agg_partials[1],
        w1, b1.reshape(1, D), w2, b2.reshape(1, D),
    )


def kernel(features, edge_index, W1a, b1a, W2a, b2a, eps1,
           W1b, b1b, W2b, b2b, eps2):
    agg1 = _sc_partial_segment_sum(features, edge_index)
    x1 = _tc_gin_mlp(features, agg1, W1a, b1a, W2a, b2a, eps1, relu_out=True)
    agg2 = _sc_partial_segment_sum(x1, edge_index)
    return _tc_gin_mlp(x1, agg2, W1b, b1b, W2b, b2b, eps2, relu_out=False)
